# baseline (speedup 1.0000x reference)
import functools

import jax
import jax.numpy as jnp
from jax import lax
from jax.experimental import pallas as pl
from jax.experimental.pallas import tpu as pltpu
from jax.experimental.pallas import tpu_sc as plsc

NSEG = 256
D = 128
N_ROWS = 100000
NC = 2
NS = 16
NW = NC * NS
CH = 256
IDC = 128
N_FULL = 12
T_BIG = 56
T_SMALL = 48
BIG_WORKERS = 20
NBUF = 3

_mesh = plsc.VectorSubcoreMesh(core_axis_name="c", subcore_axis_name="s")


@functools.partial(
    pl.kernel,
    mesh=_mesh,
    out_type=jax.ShapeDtypeStruct((NC, NSEG, D), jnp.float32),
    scratch_types=(
        [pltpu.VMEM((IDC,), jnp.int32) for _ in range(2 * NBUF)]
        + [pltpu.VMEM((CH, D), jnp.float32) for _ in range(NBUF)]
        + [
            pltpu.VMEM((T_BIG,), jnp.int32),
            pltpu.VMEM((T_SMALL,), jnp.int32),
            pltpu.VMEM((T_BIG, D), jnp.float32),
            pltpu.VMEM((T_SMALL, D), jnp.float32),
            pltpu.VMEM((NS, D), jnp.float32),
            pltpu.VMEM_SHARED((NSEG, D), jnp.float32),
        ]
        + [pltpu.SemaphoreType.DMA for _ in range(3 * NBUF)]
    ),
)
def _sc_partials(feat_hbm, ids_hbm, out_hbm, *scratch):
    ids_bufs = scratch[0:2 * NBUF]
    row_bufs = scratch[2 * NBUF:3 * NBUF]
    ids_tb, ids_ts, rows_tb, rows_ts, zbuf, acc = scratch[3 * NBUF:3 * NBUF + 6]
    sems = scratch[3 * NBUF + 6:]
    sem_i = sems[0:NBUF]
    sem_r = sems[NBUF:2 * NBUF]
    sem_s = sems[2 * NBUF:3 * NBUF]

    c = lax.axis_index("c")
    s = lax.axis_index("s")
    wid = s * NC + c
    base = pl.multiple_of(3120 * wid + 8 * jnp.minimum(wid, BIG_WORKERS), 8)

    loads = {}

    def start_load(j):
        p = j % NBUF
        off = pl.multiple_of(base + j * CH, 8)
        ci0 = pltpu.make_async_copy(
            ids_hbm.at[pl.ds(off, IDC)], ids_bufs[2 * p], sem_i[p])
        ci1 = pltpu.make_async_copy(
            ids_hbm.at[pl.ds(off + IDC, IDC)], ids_bufs[2 * p + 1], sem_i[p])
        cr = pltpu.make_async_copy(feat_hbm.at[pl.ds(off, CH), :], row_bufs[p], sem_r[p])
        ci0.start()
        ci1.start()
        cr.start()
        loads[j] = (ci0, ci1, cr)

    start_load(0)

    zero = jnp.zeros((16,), jnp.float32)
    for i in range(NS):
        for j in range(D // 16):
            zbuf[i, pl.ds(j * 16, 16)] = zero
    pltpu.sync_copy(zbuf, acc.at[pl.ds(s * NS, NS), :])
    plsc.subcore_barrier()

    for j in range(1, NBUF - 1):
        start_load(j)

    scats = {}
    for j in range(N_FULL):
        p = j % NBUF
        for cp in loads.pop(j):
            cp.wait()
        if j >= 1:
            scats.pop(j - 1).wait()
        sc0 = pltpu.make_async_copy(
            row_bufs[p].at[pl.ds(0, IDC), :], acc.at[ids_bufs[2 * p]], sem_s[p])
        sc0.start(add=True)
        sc0.wait()
        sc1 = pltpu.make_async_copy(
            row_bufs[p].at[pl.ds(IDC, IDC), :], acc.at[ids_bufs[2 * p + 1]], sem_s[p])
        sc1.start(add=True)
        scats[j] = sc1
        nxt = j + NBUF - 1
        if nxt < N_FULL:
            start_load(nxt)
    scats.pop(N_FULL - 1).wait()

    toff = pl.multiple_of(base + N_FULL * CH, 8)

    @pl.when(wid < BIG_WORKERS)
    def _big_tail():
        pltpu.sync_copy(ids_hbm.at[pl.ds(toff, T_BIG)], ids_tb)
        pltpu.sync_copy(feat_hbm.at[pl.ds(toff, T_BIG), :], rows_tb)
        pltpu.sync_copy(rows_tb, acc.at[ids_tb], add=True)

    @pl.when(wid >= BIG_WORKERS)
    def _small_tail():
        pltpu.sync_copy(ids_hbm.at[pl.ds(toff, T_SMALL)], ids_ts)
        pltpu.sync_copy(feat_hbm.at[pl.ds(toff, T_SMALL), :], rows_ts)
        pltpu.sync_copy(rows_ts, acc.at[ids_ts], add=True)

    plsc.subcore_barrier()
    pltpu.sync_copy(
        acc.at[pl.ds(s * NS, NS), :],
        out_hbm.at[c, pl.ds(s * NS, NS), :],
    )


def _combine(partials):
    def body(p_ref, o_ref):
        o_ref[...] = p_ref[0, :, :] + p_ref[1, :, :]

    return pl.pallas_call(
        body,
        out_shape=jax.ShapeDtypeStruct((NSEG, D), jnp.float32),
    )(partials)


def kernel(feat, segment_ids):
    partials = _sc_partials(feat, segment_ids.astype(jnp.int32))
    return _combine(partials)


# speedup vs baseline: 1.0733x; 1.0733x over previous
import functools

import jax
import jax.numpy as jnp
from jax import lax
from jax.experimental import pallas as pl
from jax.experimental.pallas import tpu as pltpu
from jax.experimental.pallas import tpu_sc as plsc

NSEG = 256
D = 128
N_ROWS = 100000
NC = 2
NS = 16
NW = NC * NS
CH = 128
N_FULL = 24
T_BIG = 56
T_SMALL = 48
BIG_WORKERS = 20
NBUF = 3

_mesh = plsc.VectorSubcoreMesh(core_axis_name="c", subcore_axis_name="s")


@functools.partial(
    pl.kernel,
    mesh=_mesh,
    out_type=jax.ShapeDtypeStruct((NC, NSEG, D), jnp.float32),
    scratch_types=(
        [pltpu.VMEM((CH,), jnp.int32) for _ in range(NBUF)]
        + [pltpu.VMEM((CH, D), jnp.float32) for _ in range(NBUF)]
        + [
            pltpu.VMEM((T_BIG,), jnp.int32),
            pltpu.VMEM((T_SMALL,), jnp.int32),
            pltpu.VMEM((T_BIG, D), jnp.float32),
            pltpu.VMEM((T_SMALL, D), jnp.float32),
            pltpu.VMEM((NS, D), jnp.float32),
            pltpu.VMEM_SHARED((NSEG, D), jnp.float32),
        ]
        + [pltpu.SemaphoreType.DMA for _ in range(3 * NBUF)]
    ),
)
def _sc_partials(feat_hbm, ids_hbm, out_hbm, *scratch):
    ids_bufs = scratch[0:NBUF]
    row_bufs = scratch[NBUF:2 * NBUF]
    ids_tb, ids_ts, rows_tb, rows_ts, zbuf, acc = scratch[2 * NBUF:2 * NBUF + 6]
    sems = scratch[2 * NBUF + 6:]
    sem_i = sems[0:NBUF]
    sem_r = sems[NBUF:2 * NBUF]
    sem_s = sems[2 * NBUF:3 * NBUF]

    c = lax.axis_index("c")
    s = lax.axis_index("s")
    wid = s * NC + c
    base = pl.multiple_of(3120 * wid + 8 * jnp.minimum(wid, BIG_WORKERS), 8)

    loads = {}

    def start_load(j):
        p = j % NBUF
        off = pl.multiple_of(base + j * CH, 8)
        ci = pltpu.make_async_copy(ids_hbm.at[pl.ds(off, CH)], ids_bufs[p], sem_i[p])
        cr = pltpu.make_async_copy(feat_hbm.at[pl.ds(off, CH), :], row_bufs[p], sem_r[p])
        ci.start()
        cr.start()
        loads[j] = (ci, cr)

    start_load(0)

    zero = jnp.zeros((16,), jnp.float32)
    for i in range(NS):
        for j in range(D // 16):
            zbuf[i, pl.ds(j * 16, 16)] = zero
    pltpu.sync_copy(zbuf, acc.at[pl.ds(s * NS, NS), :])
    plsc.subcore_barrier()

    for j in range(1, NBUF - 1):
        start_load(j)

    scats = {}
    for j in range(N_FULL):
        p = j % NBUF
        ci, cr = loads.pop(j)
        ci.wait()
        cr.wait()
        if j >= 1:
            scats.pop(j - 1).wait()
        sc = pltpu.make_async_copy(row_bufs[p], acc.at[ids_bufs[p]], sem_s[p])
        sc.start(add=True)
        scats[j] = sc
        nxt = j + NBUF - 1
        if nxt < N_FULL:
            start_load(nxt)
    scats.pop(N_FULL - 1).wait()

    toff = pl.multiple_of(base + N_FULL * CH, 8)

    @pl.when(wid < BIG_WORKERS)
    def _big_tail():
        pltpu.sync_copy(ids_hbm.at[pl.ds(toff, T_BIG)], ids_tb)
        pltpu.sync_copy(feat_hbm.at[pl.ds(toff, T_BIG), :], rows_tb)
        pltpu.sync_copy(rows_tb, acc.at[ids_tb], add=True)

    @pl.when(wid >= BIG_WORKERS)
    def _small_tail():
        pltpu.sync_copy(ids_hbm.at[pl.ds(toff, T_SMALL)], ids_ts)
        pltpu.sync_copy(feat_hbm.at[pl.ds(toff, T_SMALL), :], rows_ts)
        pltpu.sync_copy(rows_ts, acc.at[ids_ts], add=True)

    plsc.subcore_barrier()
    pltpu.sync_copy(
        acc.at[pl.ds(s * NS, NS), :],
        out_hbm.at[c, pl.ds(s * NS, NS), :],
    )


def _combine(partials):
    def body(p_ref, o_ref):
        o_ref[...] = p_ref[0, :, :] + p_ref[1, :, :]

    return pl.pallas_call(
        body,
        out_shape=jax.ShapeDtypeStruct((NSEG, D), jnp.float32),
    )(partials)


def kernel(feat, segment_ids):
    partials = _sc_partials(feat, segment_ids.astype(jnp.int32))
    return _combine(partials)


# speedup vs baseline: 1.0897x; 1.0152x over previous
import functools

import jax
import jax.numpy as jnp
from jax import lax
from jax.experimental import pallas as pl
from jax.experimental.pallas import tpu as pltpu
from jax.experimental.pallas import tpu_sc as plsc

NSEG = 256
D = 128
N_ROWS = 100000
NC = 2
NS = 16
NW = NC * NS
CH = 128
N_FULL = 24
T_BIG = 56
T_SMALL = 48
BIG_WORKERS = 20
NBUF = 4

_mesh = plsc.VectorSubcoreMesh(core_axis_name="c", subcore_axis_name="s")


@functools.partial(
    pl.kernel,
    mesh=_mesh,
    out_type=jax.ShapeDtypeStruct((NC, NSEG, D), jnp.float32),
    scratch_types=(
        [pltpu.VMEM((CH,), jnp.int32) for _ in range(NBUF)]
        + [pltpu.VMEM((CH, D), jnp.float32) for _ in range(NBUF)]
        + [
            pltpu.VMEM((T_BIG,), jnp.int32),
            pltpu.VMEM((T_SMALL,), jnp.int32),
            pltpu.VMEM((T_BIG, D), jnp.float32),
            pltpu.VMEM((T_SMALL, D), jnp.float32),
            pltpu.VMEM((NS, D), jnp.float32),
            pltpu.VMEM_SHARED((NSEG, D), jnp.float32),
        ]
        + [pltpu.SemaphoreType.DMA for _ in range(3 * NBUF + 1)]
    ),
)
def _sc_partials(feat_hbm, ids_hbm, out_hbm, *scratch):
    ids_bufs = scratch[0:NBUF]
    row_bufs = scratch[NBUF:2 * NBUF]
    ids_tb, ids_ts, rows_tb, rows_ts, zbuf, acc = scratch[2 * NBUF:2 * NBUF + 6]
    sems = scratch[2 * NBUF + 6:]
    sem_i = sems[0:NBUF]
    sem_r = sems[NBUF:2 * NBUF]
    sem_s = sems[2 * NBUF:3 * NBUF]
    sem_t = sems[3 * NBUF]

    c = lax.axis_index("c")
    s = lax.axis_index("s")
    wid = s * NC + c
    base = pl.multiple_of(3120 * wid + 8 * jnp.minimum(wid, BIG_WORKERS), 8)

    loads = {}

    def start_load(j):
        p = j % NBUF
        off = pl.multiple_of(base + j * CH, 8)
        ci = pltpu.make_async_copy(ids_hbm.at[pl.ds(off, CH)], ids_bufs[p], sem_i[p])
        cr = pltpu.make_async_copy(feat_hbm.at[pl.ds(off, CH), :], row_bufs[p], sem_r[p])
        ci.start()
        cr.start()
        loads[j] = (ci, cr)

    start_load(0)

    zero = jnp.zeros((16,), jnp.float32)
    for i in range(NS):
        for j in range(D // 16):
            zbuf[i, pl.ds(j * 16, 16)] = zero
    pltpu.sync_copy(zbuf, acc.at[pl.ds(s * NS, NS), :])
    plsc.subcore_barrier()

    for j in range(1, NBUF - 1):
        start_load(j)

    toff = pl.multiple_of(base + N_FULL * CH, 8)
    t_ib = pltpu.make_async_copy(ids_hbm.at[pl.ds(toff, T_BIG)], ids_tb, sem_t)
    t_rb = pltpu.make_async_copy(feat_hbm.at[pl.ds(toff, T_BIG), :], rows_tb, sem_t)
    t_is = pltpu.make_async_copy(ids_hbm.at[pl.ds(toff, T_SMALL)], ids_ts, sem_t)
    t_rs = pltpu.make_async_copy(feat_hbm.at[pl.ds(toff, T_SMALL), :], rows_ts, sem_t)

    @pl.when(wid < BIG_WORKERS)
    def _start_big_tail():
        t_ib.start()
        t_rb.start()

    @pl.when(wid >= BIG_WORKERS)
    def _start_small_tail():
        t_is.start()
        t_rs.start()

    scats = {}
    for j in range(N_FULL):
        p = j % NBUF
        ci, cr = loads.pop(j)
        ci.wait()
        cr.wait()
        if j >= 1:
            scats.pop(j - 1).wait()
        sc = pltpu.make_async_copy(row_bufs[p], acc.at[ids_bufs[p]], sem_s[p])
        sc.start(add=True)
        scats[j] = sc
        nxt = j + NBUF - 1
        if nxt < N_FULL:
            start_load(nxt)
    scats.pop(N_FULL - 1).wait()

    @pl.when(wid < BIG_WORKERS)
    def _big_tail():
        t_ib.wait()
        t_rb.wait()
        pltpu.sync_copy(rows_tb, acc.at[ids_tb], add=True)

    @pl.when(wid >= BIG_WORKERS)
    def _small_tail():
        t_is.wait()
        t_rs.wait()
        pltpu.sync_copy(rows_ts, acc.at[ids_ts], add=True)

    plsc.subcore_barrier()
    pltpu.sync_copy(
        acc.at[pl.ds(s * NS, NS), :],
        out_hbm.at[c, pl.ds(s * NS, NS), :],
    )


def _combine(partials):
    def body(p_ref, o_ref):
        o_ref[...] = p_ref[0, :, :] + p_ref[1, :, :]

    return pl.pallas_call(
        body,
        out_shape=jax.ShapeDtypeStruct((NSEG, D), jnp.float32),
    )(partials)


def kernel(feat, segment_ids):
    partials = _sc_partials(feat, segment_ids.astype(jnp.int32))
    return _combine(partials)
